# baseline (speedup 1.0000x reference)
import functools

import jax
import jax.numpy as jnp
from jax import lax
from jax.experimental import pallas as pl
from jax.experimental.pallas import tpu as pltpu
from jax.experimental.pallas import tpu_sc as plsc

N_EMB = 8192
DIM = 64
ROWS = 8192
BM = 256
N_BLOCKS = ROWS // BM
SR = 128

SC_CORES = 2
SC_SUBCORES = 16
NW = SC_CORES * SC_SUBCORES
B_PER_W = ROWS // NW
IDX_CHUNK = 128
N_CHUNKS = B_PER_W // IDX_CHUNK
DIM_PAD = 128
COMMIT = 1.0


def _dist_kernel(x_ref, w_ref, dist_ref, enc_ref, w2row_ref):
    @pl.when(pl.program_id(0) == 0)
    def _():
        w = w_ref[...]
        w2row_ref[...] = jnp.sum(w * w, axis=1)[None, :]

    for s in range(BM // SR):
        x = x_ref[pl.ds(s * SR, SR), :]
        xw_neg = lax.dot_general(-2.0 * x, w_ref[...],
                                 (((1,), (1,)), ((), ())),
                                 preferred_element_type=jnp.float32)
        x2 = jnp.sum(x * x, axis=1, keepdims=True)
        d = (x2 + w2row_ref[...]) + xw_neg
        dist_ref[pl.ds(s * SR, SR), :] = d
        enc_ref[pl.ds(s * SR, SR), :] = (
            jnp.argmin(d, axis=1).astype(jnp.int32)[:, None])


def _distances_enc_loss(flat_x, weight):
    return pl.pallas_call(
        _dist_kernel,
        grid=(N_BLOCKS,),
        in_specs=[
            pl.BlockSpec((BM, DIM), lambda i: (i, 0)),
            pl.BlockSpec((N_EMB, DIM), lambda i: (0, 0)),
        ],
        out_specs=[
            pl.BlockSpec((BM, N_EMB), lambda i: (i, 0)),
            pl.BlockSpec((BM, 1), lambda i: (i, 0)),
        ],
        out_shape=[
            jax.ShapeDtypeStruct((ROWS, N_EMB), jnp.float32),
            jax.ShapeDtypeStruct((ROWS, 1), jnp.int32),
        ],
        scratch_shapes=[pltpu.VMEM((1, N_EMB), jnp.float32)],
    )(flat_x, weight)


def _sc_gather_body(w_hbm, x_hbm, enc_hbm, out_hbm, part_hbm,
                    idx_v, rows_v, x_v, acc_v, sem):
    wid = lax.axis_index("s") * SC_CORES + lax.axis_index("c")
    base = wid * B_PER_W
    pltpu.sync_copy(enc_hbm.at[pl.ds(wid * N_CHUNKS, N_CHUNKS)], idx_v)
    pltpu.sync_copy(x_hbm.at[pl.ds(base, B_PER_W)], x_v)
    for j in range(N_CHUNKS):
        pltpu.async_copy(w_hbm.at[idx_v.at[j]],
                         rows_v.at[pl.ds(j * IDX_CHUNK, IDX_CHUNK)], sem).wait()
    pltpu.sync_copy(rows_v, out_hbm.at[pl.ds(base, B_PER_W)])

    def _row(r, acc):
        for k in range(DIM // 16):
            qv = rows_v[r, pl.ds(k * 16, 16)]
            xv = x_v[r, pl.ds(k * 16, 16)]
            t = qv - xv
            acc = acc + t * t
        return acc

    acc_v[...] = lax.fori_loop(0, B_PER_W, _row,
                               jnp.zeros((16,), jnp.float32))
    pltpu.sync_copy(acc_v, part_hbm.at[wid])


@functools.cache
def _sc_gather():
    return pl.kernel(
        _sc_gather_body,
        out_type=[
            jax.ShapeDtypeStruct((ROWS, DIM_PAD), jnp.float32),
            jax.ShapeDtypeStruct((NW, 16), jnp.float32),
        ],
        scratch_types=[
            pltpu.VMEM((N_CHUNKS, IDX_CHUNK), jnp.int32),
            pltpu.VMEM((B_PER_W, DIM_PAD), jnp.float32),
            pltpu.VMEM((B_PER_W, DIM), jnp.float32),
            pltpu.VMEM((16,), jnp.float32),
            pltpu.SemaphoreType.DMA,
        ],
        mesh=plsc.VectorSubcoreMesh(core_axis_name="c", subcore_axis_name="s"),
    )


def kernel(input, weight):
    flat_x = jnp.transpose(input, (0, 2, 3, 1)).reshape(ROWS, DIM)
    distances, enc2d = _distances_enc_loss(flat_x, weight)
    encoding_flat = enc2d.reshape(ROWS)
    weight_pad = jnp.pad(weight, ((0, 0), (0, DIM_PAD - DIM)))
    quant_pad, loss_parts = _sc_gather()(
        weight_pad, flat_x, enc2d.reshape(ROWS // IDX_CHUNK, IDX_CHUNK))
    quant_flat = quant_pad[:, :DIM]
    quantized_st = jnp.transpose(
        quant_flat.reshape(8, 32, 32, DIM), (0, 3, 1, 2))
    encoding = encoding_flat.reshape(8, 32, 32)
    loss = COMMIT * jnp.sum(loss_parts) * (1.0 / input.size)
    return (quantized_st, encoding, distances, loss)


# speedup vs baseline: 1.0197x; 1.0197x over previous
import functools

import jax
import jax.numpy as jnp
from jax import lax
from jax.experimental import pallas as pl
from jax.experimental.pallas import tpu as pltpu
from jax.experimental.pallas import tpu_sc as plsc

N_EMB = 8192
DIM = 64
ROWS = 8192
BM = 512
N_BLOCKS = ROWS // BM

SC_CORES = 2
SC_SUBCORES = 16
NW = SC_CORES * SC_SUBCORES
B_PER_W = ROWS // NW
IDX_CHUNK = 128
N_CHUNKS = B_PER_W // IDX_CHUNK
DIM_PAD = 128
COMMIT = 1.0


def _dist_kernel(x_ref, w_ref, dist_ref, enc_ref, w2row_ref):
    @pl.when(pl.program_id(0) == 0)
    def _():
        w = w_ref[...]
        w2row_ref[...] = jnp.sum(w * w, axis=1)[None, :]

    x = x_ref[...]
    xw_neg = lax.dot_general(-2.0 * x, w_ref[...], (((1,), (1,)), ((), ())),
                             preferred_element_type=jnp.float32)
    x2 = jnp.sum(x * x, axis=1, keepdims=True)
    d = (x2 + w2row_ref[...]) + xw_neg
    dist_ref[...] = d
    enc_ref[...] = jnp.argmin(d, axis=1).astype(jnp.int32)[:, None]


def _distances_enc_loss(flat_x, weight):
    return pl.pallas_call(
        _dist_kernel,
        grid=(N_BLOCKS,),
        in_specs=[
            pl.BlockSpec((BM, DIM), lambda i: (i, 0)),
            pl.BlockSpec((N_EMB, DIM), lambda i: (0, 0)),
        ],
        out_specs=[
            pl.BlockSpec((BM, N_EMB), lambda i: (i, 0)),
            pl.BlockSpec((BM, 1), lambda i: (i, 0)),
        ],
        out_shape=[
            jax.ShapeDtypeStruct((ROWS, N_EMB), jnp.float32),
            jax.ShapeDtypeStruct((ROWS, 1), jnp.int32),
        ],
        scratch_shapes=[pltpu.VMEM((1, N_EMB), jnp.float32)],
    )(flat_x, weight)


def _sc_gather_body(w_hbm, x_hbm, enc_hbm, out_hbm, part_hbm,
                    idx_v, rows_v, x_v, acc_v, sem):
    wid = lax.axis_index("s") * SC_CORES + lax.axis_index("c")
    base = wid * B_PER_W
    pltpu.sync_copy(enc_hbm.at[pl.ds(wid * N_CHUNKS, N_CHUNKS)], idx_v)
    copies = [
        pltpu.async_copy(w_hbm.at[idx_v.at[j]],
                         rows_v.at[pl.ds(j * IDX_CHUNK, IDX_CHUNK)], sem)
        for j in range(N_CHUNKS)
    ]
    pltpu.sync_copy(x_hbm.at[pl.ds(base, B_PER_W)], x_v)
    for c in copies:
        c.wait()
    pltpu.sync_copy(rows_v, out_hbm.at[pl.ds(base, B_PER_W)])

    def _row(r, acc):
        for k in range(DIM // 16):
            qv = rows_v[r, pl.ds(k * 16, 16)]
            xv = x_v[r, pl.ds(k * 16, 16)]
            t = qv - xv
            acc = acc + t * t
        return acc

    acc_v[...] = lax.fori_loop(0, B_PER_W, _row,
                               jnp.zeros((16,), jnp.float32))
    pltpu.sync_copy(acc_v, part_hbm.at[wid])


@functools.cache
def _sc_gather():
    return pl.kernel(
        _sc_gather_body,
        out_type=[
            jax.ShapeDtypeStruct((ROWS, DIM_PAD), jnp.float32),
            jax.ShapeDtypeStruct((NW, 16), jnp.float32),
        ],
        scratch_types=[
            pltpu.VMEM((N_CHUNKS, IDX_CHUNK), jnp.int32),
            pltpu.VMEM((B_PER_W, DIM_PAD), jnp.float32),
            pltpu.VMEM((B_PER_W, DIM), jnp.float32),
            pltpu.VMEM((16,), jnp.float32),
            pltpu.SemaphoreType.DMA,
        ],
        mesh=plsc.VectorSubcoreMesh(core_axis_name="c", subcore_axis_name="s"),
    )


def kernel(input, weight):
    flat_x = jnp.transpose(input, (0, 2, 3, 1)).reshape(ROWS, DIM)
    distances, enc2d = _distances_enc_loss(flat_x, weight)
    encoding_flat = enc2d.reshape(ROWS)
    weight_pad = jnp.pad(weight, ((0, 0), (0, DIM_PAD - DIM)))
    quant_pad, loss_parts = _sc_gather()(
        weight_pad, flat_x, enc2d.reshape(ROWS // IDX_CHUNK, IDX_CHUNK))
    quant_flat = quant_pad[:, :DIM]
    quantized_st = jnp.transpose(
        quant_flat.reshape(8, 32, 32, DIM), (0, 3, 1, 2))
    encoding = encoding_flat.reshape(8, 32, 32)
    loss = COMMIT * jnp.sum(loss_parts) * (1.0 / input.size)
    return (quantized_st, encoding, distances, loss)
